# baseline (speedup 1.0000x reference)
import jax
import jax.numpy as jnp
from jax.experimental import pallas as pl
from jax.experimental.pallas import tpu as pltpu


def _head_kernel(x_ref, w_ref, b_ref, o_ref, wv_ref, wp_ref, sem):
    tb, n, p, f = x_ref.shape
    h = o_ref.shape[-1]
    nf = p * f
    h_pad = wp_ref.shape[-1]

    @pl.when(pl.program_id(1) == 0)
    def _():
        copy = pltpu.make_async_copy(w_ref, wv_ref, sem)
        copy.start()
        copy.wait()
        w3 = wv_ref[...].reshape(f, p, h_pad)
        wp_ref[...] = (
            w3.transpose(1, 0, 2).reshape(nf, h_pad).astype(jnp.bfloat16)
        )

    xf = x_ref[...].reshape(tb * n, nf).astype(jnp.bfloat16)
    acc = jnp.dot(xf, wp_ref[...], preferred_element_type=jnp.float32)
    res = acc + b_ref[...]
    o_ref[...] = res[:, :h].reshape(tb, n, h).astype(o_ref.dtype)


def kernel(x, w_t, b):
    H = 336
    B, N, F, P = x.shape
    nf = F * P
    nf_w, H_pad = w_t.shape
    out_dtype = x.dtype

    xt = jnp.transpose(x, (0, 1, 3, 2))

    tb = 32
    while B % (2 * tb) != 0:
        tb //= 2
    grid_j = B // (2 * tb)

    need = (2 * tb * N * P * F * 4
            + nf * H_pad * 4
            + nf * H_pad * 2
            + 8 * H_pad * 4
            + 2 * tb * 8 * ((H + 127) // 128 * 128) * 4)
    vmem_limit = int(min(need + (8 << 20), 100 << 20))

    return pl.pallas_call(
        _head_kernel,
        out_shape=jax.ShapeDtypeStruct((B, N, H), out_dtype),
        grid=(2, grid_j),
        in_specs=[
            pl.BlockSpec((tb, N, P, F),
                         lambda c, j: (c * grid_j + j, 0, 0, 0)),
            pl.BlockSpec(memory_space=pltpu.MemorySpace.HBM),
            pl.BlockSpec((1, H_pad), lambda c, j: (0, 0)),
        ],
        out_specs=pl.BlockSpec((tb, N, H), lambda c, j: (c * grid_j + j, 0, 0)),
        scratch_shapes=[
            pltpu.VMEM((nf, H_pad), jnp.float32),
            pltpu.VMEM((nf, H_pad), jnp.bfloat16),
            pltpu.SemaphoreType.DMA,
        ],
        compiler_params=pltpu.CompilerParams(
            dimension_semantics=("parallel", "arbitrary"),
            vmem_limit_bytes=vmem_limit,
        ),
    )(xt, w_t, b)


# speedup vs baseline: 1.0541x; 1.0541x over previous
import jax
import jax.numpy as jnp
from jax.experimental import pallas as pl
from jax.experimental.pallas import tpu as pltpu


def _head_kernel(x_ref, w_ref, b_ref, o_ref, wp_ref):
    tb, n, p, f = x_ref.shape
    h = o_ref.shape[-1]
    nf = p * f
    h_pad = w_ref.shape[-1]

    @pl.when(pl.program_id(1) == 0)
    def _():
        w3 = w_ref[...].reshape(f, p, h_pad)
        wp_ref[...] = (
            w3.transpose(1, 0, 2).reshape(nf, h_pad).astype(jnp.bfloat16)
        )

    xf = x_ref[...].reshape(tb * n, nf).astype(jnp.bfloat16)
    acc = jnp.dot(xf, wp_ref[...], preferred_element_type=jnp.float32)
    res = acc + b_ref[...]
    o_ref[...] = res[:, :h].reshape(tb, n, h).astype(o_ref.dtype)


def kernel(x, w_t, b):
    H = 336
    B, N, F, P = x.shape
    nf = F * P
    nf_w, H_pad = w_t.shape
    out_dtype = x.dtype

    xt = jnp.transpose(x, (0, 1, 3, 2))

    tb = 64
    while B % (2 * tb) != 0:
        tb //= 2
    grid_j = B // (2 * tb)

    need = (2 * tb * N * P * F * 4
            + nf * H_pad * 4
            + nf * H_pad * 2
            + 8 * H_pad * 4
            + 2 * tb * 8 * ((H + 127) // 128 * 128) * 4)
    vmem_limit = int(min(need + (8 << 20), 100 << 20))

    return pl.pallas_call(
        _head_kernel,
        out_shape=jax.ShapeDtypeStruct((B, N, H), out_dtype),
        grid=(2, grid_j),
        in_specs=[
            pl.BlockSpec((tb, N, P, F),
                         lambda c, j: (c * grid_j + j, 0, 0, 0)),
            pl.BlockSpec((nf, H_pad), lambda c, j: (0, 0)),
            pl.BlockSpec((1, H_pad), lambda c, j: (0, 0)),
        ],
        out_specs=pl.BlockSpec((tb, N, H), lambda c, j: (c * grid_j + j, 0, 0)),
        scratch_shapes=[
            pltpu.VMEM((nf, H_pad), jnp.bfloat16),
        ],
        compiler_params=pltpu.CompilerParams(
            dimension_semantics=("parallel", "arbitrary"),
            vmem_limit_bytes=vmem_limit,
        ),
    )(xt, w_t, b)


# speedup vs baseline: 1.0611x; 1.0067x over previous
import jax
import jax.numpy as jnp
from jax.experimental import pallas as pl
from jax.experimental.pallas import tpu as pltpu


def _head_kernel(x_ref, w_ref, b_ref, o_ref, wp_ref, bv_ref, sem):
    tb, n, p, f = x_ref.shape
    h = o_ref.shape[-1]
    nf = p * f
    h_pad = w_ref.shape[-1]

    @pl.when(pl.program_id(1) == 0)
    def _():
        bcopy = pltpu.make_async_copy(b_ref, bv_ref, sem)
        bcopy.start()
        w3 = w_ref[...].reshape(f, p, h_pad)
        wp_ref[...] = (
            w3.transpose(1, 0, 2).reshape(nf, h_pad).astype(jnp.bfloat16)
        )

    @pl.when(pl.program_id(1) == 0)
    def _():
        pltpu.make_async_copy(b_ref, bv_ref, sem).wait()

    xf = x_ref[...].reshape(tb * n, nf).astype(jnp.bfloat16)
    acc = jnp.dot(xf, wp_ref[...], preferred_element_type=jnp.float32)
    res = acc + bv_ref[...]
    o_ref[...] = res[:, :h].reshape(tb, n, h).astype(o_ref.dtype)


def kernel(x, w_t, b):
    H = 336
    B, N, F, P = x.shape
    nf = F * P
    nf_w, H_pad = w_t.shape
    out_dtype = x.dtype

    xt = jnp.transpose(x, (0, 1, 3, 2))

    tb = 64
    while B % (2 * tb) != 0:
        tb //= 2
    grid_j = B // (2 * tb)

    need = (2 * tb * N * P * F * 4
            + nf * H_pad * 4
            + nf * H_pad * 2
            + 8 * H_pad * 4
            + 2 * tb * 8 * ((H + 127) // 128 * 128) * 4)
    vmem_limit = int(min(need + (8 << 20), 100 << 20))

    return pl.pallas_call(
        _head_kernel,
        out_shape=jax.ShapeDtypeStruct((B, N, H), out_dtype),
        grid=(2, grid_j),
        in_specs=[
            pl.BlockSpec((tb, N, P, F),
                         lambda c, j: (c * grid_j + j, 0, 0, 0)),
            pl.BlockSpec((nf, H_pad), lambda c, j: (0, 0)),
            pl.BlockSpec(memory_space=pltpu.MemorySpace.HBM),
        ],
        out_specs=pl.BlockSpec((tb, N, H), lambda c, j: (c * grid_j + j, 0, 0)),
        scratch_shapes=[
            pltpu.VMEM((nf, H_pad), jnp.bfloat16),
            pltpu.VMEM((1, H_pad), jnp.float32),
            pltpu.SemaphoreType.DMA,
        ],
        compiler_params=pltpu.CompilerParams(
            dimension_semantics=("parallel", "arbitrary"),
            vmem_limit_bytes=vmem_limit,
        ),
    )(xt, w_t, b)
